# Initial kernel scaffold; baseline (speedup 1.0000x reference)
import functools

import jax
import jax.numpy as jnp
from jax import lax
from jax.experimental import pallas as pl
from jax.experimental.pallas import tpu as pltpu
from jax.experimental.pallas import tpu_sc as plsc

_info = plsc.get_sparse_core_info()
_NC = _info.num_cores
_NS = _info.num_subcores
_NW = _NC * _NS
_LANE = 128


@functools.lru_cache(maxsize=None)
def _make_gather(n_idx, dim, chunk_rows):
    rows_per_w = n_idx // (_NW * _LANE)
    n_chunks = rows_per_w // chunk_rows
    chunk_idx = chunk_rows * _LANE

    mesh = plsc.VectorSubcoreMesh(core_axis_name="c", subcore_axis_name="s")

    @functools.partial(
        pl.kernel,
        mesh=mesh,
        out_type=jax.ShapeDtypeStruct((n_idx, dim), jnp.float32),
        scratch_types=[
            pltpu.VMEM((chunk_rows, _LANE), jnp.int32),
            pltpu.VMEM((chunk_idx, dim), jnp.float32),
            pltpu.SemaphoreType.DMA,
        ],
    )
    def gather_kernel(idx_hbm, table_hbm, out_hbm, idx_v, rows_v, sem):
        wid = lax.axis_index("s") * _NC + lax.axis_index("c")
        row_base = wid * rows_per_w

        def body(ci, carry):
            row_off = row_base + ci * chunk_rows
            pltpu.sync_copy(idx_hbm.at[pl.ds(row_off, chunk_rows)], idx_v)
            copies = [
                pltpu.async_copy(
                    table_hbm.at[idx_v.at[j]],
                    rows_v.at[pl.ds(j * _LANE, _LANE)],
                    sem,
                )
                for j in range(chunk_rows)
            ]
            for c in copies:
                c.wait()
            pltpu.sync_copy(rows_v, out_hbm.at[pl.ds(row_off * _LANE, chunk_idx)])
            return carry

        lax.fori_loop(0, n_chunks, body, 0)

    return gather_kernel


def kernel(news_ids, table):
    batch, hist = news_ids.shape
    _, dim = table.shape
    n_idx = batch * hist
    idx2d = news_ids.reshape(n_idx // _LANE, _LANE)
    out = _make_gather(n_idx, dim, 8)(idx2d, table)
    return out.reshape(batch, hist, dim)


# speedup vs baseline: 1.0937x; 1.0937x over previous
import functools

import jax
import jax.numpy as jnp
from jax import lax
from jax.experimental import pallas as pl
from jax.experimental.pallas import tpu as pltpu
from jax.experimental.pallas import tpu_sc as plsc

_info = plsc.get_sparse_core_info()
_NC = _info.num_cores
_NS = _info.num_subcores
_NW = _NC * _NS
_LANE = 128


@functools.lru_cache(maxsize=None)
def _make_gather(n_idx, dim, chunk_rows):
    rows_per_w = n_idx // (_NW * _LANE)
    n_chunks = rows_per_w // chunk_rows
    chunk_idx = chunk_rows * _LANE

    mesh = plsc.VectorSubcoreMesh(core_axis_name="c", subcore_axis_name="s")

    @functools.partial(
        pl.kernel,
        mesh=mesh,
        out_type=jax.ShapeDtypeStruct((n_idx, dim), jnp.float32),
        scratch_types=[
            pltpu.VMEM((chunk_rows, _LANE), jnp.int32),
            pltpu.VMEM((chunk_idx, dim), jnp.float32),
            pltpu.SemaphoreType.DMA,
        ],
        compiler_params=pltpu.CompilerParams(use_tc_tiling_on_sc=False),
    )
    def gather_kernel(idx_hbm, table_hbm, out_hbm, idx_v, rows_v, sem):
        wid = lax.axis_index("s") * _NC + lax.axis_index("c")
        row_base = wid * rows_per_w

        def body(ci, carry):
            row_off = row_base + ci * chunk_rows
            pltpu.sync_copy(idx_hbm.at[pl.ds(row_off, chunk_rows)], idx_v)
            copies = [
                pltpu.async_copy(
                    table_hbm.at[idx_v.at[j]],
                    rows_v.at[pl.ds(j * _LANE, _LANE)],
                    sem,
                )
                for j in range(chunk_rows)
            ]
            for c in copies:
                c.wait()
            pltpu.sync_copy(rows_v, out_hbm.at[pl.ds(row_off * _LANE, chunk_idx)])
            return carry

        lax.fori_loop(0, n_chunks, body, 0)

    return gather_kernel


def kernel(news_ids, table):
    batch, hist = news_ids.shape
    _, dim = table.shape
    n_idx = batch * hist
    idx2d = news_ids.reshape(n_idx // _LANE, _LANE)
    out = _make_gather(n_idx, dim, 8)(idx2d, table)
    return out.reshape(batch, hist, dim)


# speedup vs baseline: 1.1128x; 1.0175x over previous
import functools

import jax
import jax.numpy as jnp
from jax import lax
from jax.experimental import pallas as pl
from jax.experimental.pallas import tpu as pltpu
from jax.experimental.pallas import tpu_sc as plsc

_info = plsc.get_sparse_core_info()
_NC = _info.num_cores
_NS = _info.num_subcores
_NW = _NC * _NS
_LANE = 128
_CHUNK_ROWS = 4
_NBUF = 5


@functools.lru_cache(maxsize=None)
def _make_gather(n_idx, dim):
    rows_per_w = n_idx // (_NW * _LANE)
    n_chunks = rows_per_w // _CHUNK_ROWS
    chunk_idx = _CHUNK_ROWS * _LANE
    n_steady = n_chunks - _NBUF
    assert n_chunks * _CHUNK_ROWS == rows_per_w and n_steady % _NBUF == 0

    mesh = plsc.VectorSubcoreMesh(core_axis_name="c", subcore_axis_name="s")

    @functools.partial(
        pl.kernel,
        mesh=mesh,
        out_type=jax.ShapeDtypeStruct((n_idx, dim), jnp.float32),
        scratch_types=[
            pltpu.VMEM((rows_per_w, _LANE), jnp.int32),
            *[pltpu.VMEM((chunk_idx, dim), jnp.float32) for _ in range(_NBUF)],
            *[pltpu.SemaphoreType.DMA for _ in range(2 * _NBUF)],
        ],
        compiler_params=pltpu.CompilerParams(use_tc_tiling_on_sc=False),
    )
    def gather_kernel(idx_hbm, table_hbm, out_hbm, idx_v, *bufs_and_sems):
        rows_v = bufs_and_sems[:_NBUF]
        gsem = bufs_and_sems[_NBUF:2 * _NBUF]
        ssem = bufs_and_sems[2 * _NBUF:]
        wid = lax.axis_index("s") * _NC + lax.axis_index("c")
        row_base = wid * rows_per_w
        idx_base = row_base * _LANE

        pltpu.sync_copy(idx_hbm.at[pl.ds(row_base, rows_per_w)], idx_v)

        def fire_gathers(ci, b):
            for j in range(_CHUNK_ROWS):
                pltpu.async_copy(
                    table_hbm.at[idx_v.at[ci * _CHUNK_ROWS + j]],
                    rows_v[b].at[pl.ds(j * _LANE, _LANE)],
                    gsem[b],
                )

        def drain_gathers(b):
            pltpu.make_async_copy(
                out_hbm.at[pl.ds(0, chunk_idx)], rows_v[b], gsem[b]
            ).wait()

        def fire_store(ci, b):
            pltpu.async_copy(
                rows_v[b],
                out_hbm.at[pl.ds(idx_base + ci * chunk_idx, chunk_idx)],
                ssem[b],
            )

        def wait_store(ci, b):
            pltpu.make_async_copy(
                rows_v[b],
                out_hbm.at[pl.ds(idx_base + ci * chunk_idx, chunk_idx)],
                ssem[b],
            ).wait()

        for b in range(_NBUF):
            fire_gathers(b, b)

        def body(i, carry):
            t = i * _NBUF
            for b in range(_NBUF):
                ci = t + b
                drain_gathers(b)
                fire_store(ci, b)
                wait_store(ci, b)
                fire_gathers(ci + _NBUF, b)
            return carry

        lax.fori_loop(0, n_steady // _NBUF, body, 0)

        for b in range(_NBUF):
            ci = n_steady + b
            drain_gathers(b)
            fire_store(ci, b)
            wait_store(ci, b)

    return gather_kernel


def kernel(news_ids, table):
    batch, hist = news_ids.shape
    _, dim = table.shape
    n_idx = batch * hist
    idx2d = news_ids.reshape(n_idx // _LANE, _LANE)
    out = _make_gather(n_idx, dim)(idx2d, table)
    return out.reshape(batch, hist, dim)


# speedup vs baseline: 1.8070x; 1.6238x over previous
import functools

import jax
import jax.numpy as jnp
from jax import lax
from jax.experimental import pallas as pl
from jax.experimental.pallas import tpu as pltpu
from jax.experimental.pallas import tpu_sc as plsc

_info = plsc.get_sparse_core_info()
_NC = _info.num_cores
_NS = _info.num_subcores
_NW = _NC * _NS
_CHUNK = 8
_NBUF = 4


@functools.lru_cache(maxsize=None)
def _make_gather(batch, hist, dim):
    rows_per_w = batch // _NW
    n_chunks = rows_per_w // _CHUNK
    n_steady = n_chunks - _NBUF
    assert n_chunks * _CHUNK == rows_per_w and n_steady % _NBUF == 0

    mesh = plsc.VectorSubcoreMesh(core_axis_name="c", subcore_axis_name="s")

    @functools.partial(
        pl.kernel,
        mesh=mesh,
        out_type=jax.ShapeDtypeStruct((batch, hist, dim), jnp.float32),
        scratch_types=[
            pltpu.VMEM((rows_per_w, hist), jnp.int32),
            *[pltpu.VMEM((_CHUNK, hist, dim), jnp.float32) for _ in range(_NBUF)],
            *[pltpu.SemaphoreType.DMA for _ in range(2 * _NBUF)],
        ],
        compiler_params=pltpu.CompilerParams(use_tc_tiling_on_sc=False),
    )
    def gather_kernel(idx_hbm, table_hbm, out_hbm, idx_v, *bufs_and_sems):
        rows_v = bufs_and_sems[:_NBUF]
        gsem = bufs_and_sems[_NBUF:2 * _NBUF]
        ssem = bufs_and_sems[2 * _NBUF:]
        wid = lax.axis_index("s") * _NC + lax.axis_index("c")
        row_base = wid * rows_per_w

        pltpu.sync_copy(idx_hbm.at[pl.ds(row_base, rows_per_w)], idx_v)

        def fire_gathers(ci, b):
            for r in range(_CHUNK):
                pltpu.async_copy(
                    table_hbm.at[idx_v.at[ci * _CHUNK + r]],
                    rows_v[b].at[r],
                    gsem[b],
                )

        def drain_gathers(b):
            pltpu.make_async_copy(
                out_hbm.at[pl.ds(0, _CHUNK)], rows_v[b], gsem[b]
            ).wait()

        def fire_store(ci, b):
            pltpu.async_copy(
                rows_v[b],
                out_hbm.at[pl.ds(row_base + ci * _CHUNK, _CHUNK)],
                ssem[b],
            )

        def wait_store(ci, b):
            pltpu.make_async_copy(
                rows_v[b],
                out_hbm.at[pl.ds(row_base + ci * _CHUNK, _CHUNK)],
                ssem[b],
            ).wait()

        for b in range(_NBUF):
            fire_gathers(b, b)

        def body(i, carry):
            t = i * _NBUF
            for b in range(_NBUF):
                ci = t + b
                drain_gathers(b)
                fire_store(ci, b)
                wait_store(ci, b)
                fire_gathers(ci + _NBUF, b)
            return carry

        lax.fori_loop(0, n_steady // _NBUF, body, 0)

        for b in range(_NBUF):
            ci = n_steady + b
            drain_gathers(b)
            fire_store(ci, b)
            wait_store(ci, b)

    return gather_kernel


def kernel(news_ids, table):
    batch, hist = news_ids.shape
    _, dim = table.shape
    return _make_gather(batch, hist, dim)(news_ids, table)
